# baseline (speedup 1.0000x reference)
import functools

import jax
import jax.numpy as jnp
from jax import lax
from jax.experimental import pallas as pl
from jax.experimental.pallas import tpu as pltpu
from jax.experimental.pallas import tpu_sc as plsc

NC = 2
NS = 16
NW = NC * NS
LANES = 16


def _make_kernel(B, L, V, D):
    rows_total = B * L
    rows_per_w = rows_total // NW
    bpw = B // NW
    CH = 512
    chunks_per_batch = L // CH
    vecs_per_row = D // LANES

    mesh = plsc.VectorSubcoreMesh(core_axis_name="c", subcore_axis_name="s")

    @functools.partial(
        pl.kernel,
        mesh=mesh,
        compiler_params=pltpu.CompilerParams(use_tc_tiling_on_sc=False),
        out_type=jax.ShapeDtypeStruct((rows_total, D), jnp.float32),
        scratch_types=[
            pltpu.VMEM((CH,), jnp.int32),
            pltpu.VMEM((CH, D), jnp.float32),
            pltpu.VMEM((CH, D), jnp.float32),
            pltpu.SemaphoreType.DMA,
        ],
    )
    def k(idx_hbm, wt_hbm, pt_hbm, out_hbm, idx_v, rows_v, pos_v, sem):
        cid = lax.axis_index("c")
        sid = lax.axis_index("s")
        wid = sid * NC + cid

        for c in range(chunks_per_batch):
            pltpu.sync_copy(pt_hbm.at[pl.ds(c * CH, CH)], pos_v)
            for bi in range(bpw):
                b = wid * bpw + bi
                base = b * L + c * CH
                pltpu.sync_copy(idx_hbm.at[pl.ds(base, CH)], idx_v)
                pltpu.async_copy(wt_hbm.at[idx_v], rows_v, sem).wait()

                def add_row(r, _):
                    for v in range(vecs_per_row):
                        sl = pl.ds(v * LANES, LANES)
                        rows_v[r, sl] = rows_v[r, sl] + pos_v[r, sl]
                    return 0

                lax.fori_loop(0, CH, add_row, 0)
                pltpu.sync_copy(rows_v, out_hbm.at[pl.ds(base, CH)])

    return k


def kernel(inputs, word_table, pos_table):
    B, L = inputs.shape
    V, D = word_table.shape
    flat_idx = inputs.reshape(B * L).astype(jnp.int32)
    k = _make_kernel(B, L, V, D)
    out = k(flat_idx, word_table, pos_table)
    return out.reshape(B, L, D)


# speedup vs baseline: 1.4216x; 1.4216x over previous
import functools

import jax
import jax.numpy as jnp
from jax import lax
from jax.experimental import pallas as pl
from jax.experimental.pallas import tpu as pltpu
from jax.experimental.pallas import tpu_sc as plsc

NC = 2
NS = 16
NW = NC * NS
LANES = 16


def _make_kernel(B, L, V, D):
    rows_total = B * L
    bpw = B // NW
    CH = 256
    chunks_per_batch = L // CH
    n_chunks = bpw * chunks_per_batch
    vecs_per_row = D // LANES

    mesh = plsc.VectorSubcoreMesh(core_axis_name="c", subcore_axis_name="s")

    @functools.partial(
        pl.kernel,
        mesh=mesh,
        out_type=jax.ShapeDtypeStruct((rows_total, D), jnp.float32),
        scratch_types=[
            pltpu.VMEM((2, CH), jnp.int32),
            pltpu.VMEM((2, CH, D), jnp.float32),
            pltpu.VMEM((CH, D), jnp.float32),
            pltpu.SemaphoreType.DMA,
            pltpu.SemaphoreType.DMA,
        ],
    )
    def k(idx_hbm, wt_hbm, pt_hbm, out_hbm, idx_v, rows_v, pos_v, gsem, psem):
        cid = lax.axis_index("c")
        sid = lax.axis_index("s")
        wid = sid * NC + cid

        def chunk_base(c):
            bi = c // chunks_per_batch
            lc = c % chunks_per_batch
            return (wid * bpw + bi) * L + lc * CH

        def fire(c, buf):
            base = chunk_base(c)
            pltpu.sync_copy(idx_hbm.at[pl.ds(base, CH)], idx_v.at[buf])

            def issue(g, _):
                r0 = g * LANES
                ivec = idx_v[buf, pl.ds(r0, LANES)]
                for j in range(LANES):
                    pltpu.async_copy(
                        wt_hbm.at[ivec[j]], rows_v.at[buf, r0 + j], gsem
                    )
                return 0

            lax.fori_loop(0, CH // LANES, issue, 0)

        def drain_add_store(c, buf):
            base = chunk_base(c)
            lc = c % chunks_per_batch
            pltpu.async_copy(pt_hbm.at[pl.ds(lc * CH, CH)], pos_v, psem).wait()

            pltpu.make_async_copy(
                wt_hbm.at[pl.ds(0, CH)], rows_v.at[buf], gsem
            ).wait()

            def add_row(r, _):
                for v in range(vecs_per_row):
                    sl = pl.ds(v * LANES, LANES)
                    rows_v[buf, r, sl] = rows_v[buf, r, sl] + pos_v[r, sl]
                return 0

            lax.fori_loop(0, CH, add_row, 0, unroll=4)
            pltpu.sync_copy(rows_v.at[buf], out_hbm.at[pl.ds(base, CH)])

        fire(0, 0)
        for c in range(n_chunks):
            if c + 1 < n_chunks:
                fire(c + 1, (c + 1) % 2)
            drain_add_store(c, c % 2)

    return k


def kernel(inputs, word_table, pos_table):
    B, L = inputs.shape
    V, D = word_table.shape
    flat_idx = inputs.reshape(B * L).astype(jnp.int32)
    k = _make_kernel(B, L, V, D)
    out = k(flat_idx, word_table, pos_table)
    return out.reshape(B, L, D)
